# baseline (speedup 1.0000x reference)
import functools

import jax
import jax.numpy as jnp
from jax import lax
from jax.experimental import pallas as pl
from jax.experimental.pallas import tpu as pltpu
from jax.experimental.pallas import tpu_sc as plsc

NUM_SEGMENTS = 10000
ROWS = 320000
D = 128
NC = 2
NS = 16
GROUP = 128
NGROUPS = ROWS // GROUP
GROUPS_PER_CORE = NGROUPS // NC
BASE_GROUPS = GROUPS_PER_CORE // NS
EXTRA_TILES = GROUPS_PER_CORE % NS
SEG_PER_TILE = 624
SEG_TAIL = NUM_SEGMENTS - NS * SEG_PER_TILE

_mesh = plsc.VectorSubcoreMesh(
    core_axis_name="c", subcore_axis_name="s", num_cores=NC, num_subcores=NS
)


@functools.partial(
    pl.kernel,
    out_type=jax.ShapeDtypeStruct((NC, NUM_SEGMENTS, D), jnp.float32),
    mesh=_mesh,
    scratch_types=[
        pltpu.VMEM((8, GROUP), jnp.int32),
        pltpu.VMEM((3, GROUP, D), jnp.float32),
        pltpu.VMEM_SHARED((NUM_SEGMENTS, D), jnp.float32),
        pltpu.SemaphoreType.DMA,
        pltpu.SemaphoreType.DMA((3,)),
    ],
)
def _segment_sum_sc(x_hbm, idxp_hbm, zeros_hbm, part_hbm, idx_v, x_v, acc,
                    isem, xsem):
    c = lax.axis_index("c")
    s = lax.axis_index("s")

    n_groups = BASE_GROUPS + jnp.where(s < EXTRA_TILES, 1, 0)
    g0 = GROUPS_PER_CORE * c + BASE_GROUPS * s + jnp.minimum(s, EXTRA_TILES)

    off = lax.rem(g0, 8)
    ga = g0 - off
    n_chunks = (off + n_groups + 7) // 8

    def start_x(i, b):
        pltpu.async_copy(
            x_hbm.at[pl.ds((g0 + i) * GROUP, GROUP)], x_v.at[b], xsem.at[b]
        )

    def wait_x(b):
        pltpu.make_async_copy(
            x_hbm.at[pl.ds(0, GROUP)], x_v.at[b], xsem.at[b]
        ).wait()

    def start_idx(k):
        o = pl.multiple_of(ga + 8 * k, 8)
        pltpu.async_copy(idxp_hbm.at[pl.ds(o, 8)], idx_v, isem)

    def wait_idx():
        pltpu.make_async_copy(idxp_hbm.at[pl.ds(0, 8)], idx_v, isem).wait()

    start_idx(0)
    start_x(0, 0)
    start_x(1, 1)
    start_x(2, 2)

    pltpu.sync_copy(
        zeros_hbm.at[pl.ds(s * SEG_PER_TILE, SEG_PER_TILE)],
        acc.at[pl.ds(s * SEG_PER_TILE, SEG_PER_TILE)],
    )

    @pl.when(s == NS - 1)
    def _zero_tail():
        pltpu.sync_copy(
            zeros_hbm.at[pl.ds(NS * SEG_PER_TILE, SEG_TAIL)],
            acc.at[pl.ds(NS * SEG_PER_TILE, SEG_TAIL)],
        )

    plsc.subcore_barrier()

    def outer(k, carry):
        wait_idx()
        lo = jnp.maximum(8 * k - off, 0)
        hi = jnp.minimum(8 * k + 8 - off, n_groups)

        def inner(i, carry2):
            b = lax.rem(i, 3)
            wait_x(b)
            pltpu.sync_copy(
                x_v.at[b], acc.at[idx_v.at[off + i - 8 * k]], add=True
            )

            @pl.when(i + 3 < n_groups)
            def _refill():
                start_x(i + 3, b)

            return carry2

        lax.fori_loop(lo, hi, inner, 0)

        @pl.when(k + 1 < n_chunks)
        def _next_idx():
            start_idx(k + 1)

        return carry

    lax.fori_loop(0, n_chunks, outer, 0)
    plsc.subcore_barrier()

    pltpu.sync_copy(
        acc.at[pl.ds(s * SEG_PER_TILE, SEG_PER_TILE)],
        part_hbm.at[c, pl.ds(s * SEG_PER_TILE, SEG_PER_TILE)],
    )

    @pl.when(s == NS - 1)
    def _write_tail():
        pltpu.sync_copy(
            acc.at[pl.ds(NS * SEG_PER_TILE, SEG_TAIL)],
            part_hbm.at[c, pl.ds(NS * SEG_PER_TILE, SEG_TAIL)],
        )


ROWS_PER_BLOCK = 2000


def _combine_body(scale_ref, part_ref, out_ref):
    out_ref[...] = (part_ref[0] + part_ref[1]) * scale_ref[0]


_combine = pl.pallas_call(
    _combine_body,
    grid=(NUM_SEGMENTS // ROWS_PER_BLOCK,),
    in_specs=[
        pl.BlockSpec(memory_space=pltpu.SMEM),
        pl.BlockSpec((NC, ROWS_PER_BLOCK, D), lambda i: (0, i, 0)),
    ],
    out_specs=pl.BlockSpec((ROWS_PER_BLOCK, D), lambda i: (i, 0)),
    out_shape=jax.ShapeDtypeStruct((NUM_SEGMENTS, D), jnp.float32),
)


PAD_GROUPS = 2560


def kernel(x, index, dim_size):
    idx32 = index.astype(jnp.int32)
    idx_pad = jnp.concatenate(
        [idx32, jnp.zeros((PAD_GROUPS * GROUP - ROWS,), jnp.int32)]
    ).reshape(PAD_GROUPS, GROUP)
    zeros = jnp.zeros((NUM_SEGMENTS, D), jnp.float32)
    partials = _segment_sum_sc(x, idx_pad, zeros)
    scale = jnp.asarray(dim_size == NUM_SEGMENTS, jnp.float32).reshape((1,))
    return _combine(scale, partials)
